# Initial kernel scaffold; baseline (speedup 1.0000x reference)
import jax
import jax.numpy as jnp
from jax.experimental import pallas as pl


def kernel(x, e, u, edge_index, batch, W0, b0, W1, b1, W2, b2, ln_scale, ln_bias):
    raise NotImplementedError("write your pallas kernel here")


# speedup vs baseline: 1.0101x; 1.0101x over previous
import functools

import jax
import jax.numpy as jnp
from jax import lax
from jax.experimental import pallas as pl
from jax.experimental.pallas import tpu as pltpu

N_NODES = 10000
N_EDGES = 320000
D_FEAT = 128
D_EDGE = 16
N_GRAPHS = 16
D_GLOBAL = 32
H1 = 256
H2 = 256
N_OUT = 128

BN = 1000


def _mlp_body(x_ref, s_ref, bat_ref, u_ref, w0x_ref, w0e_ref, w0u_ref,
              b0_ref, w1_ref, b1_ref, w2_ref, b2_ref, g_ref, bb_ref, o_ref):
    f32 = jnp.float32
    uw = jnp.dot(u_ref[...], w0u_ref[...], preferred_element_type=f32)
    onehot = (bat_ref[...] == lax.broadcasted_iota(jnp.int32, (BN, N_GRAPHS), 1)
              ).astype(f32)
    h = jnp.dot(x_ref[...], w0x_ref[...], preferred_element_type=f32)
    h = h + jnp.dot(s_ref[...], w0e_ref[...], preferred_element_type=f32)
    h = h + jnp.dot(onehot, uw, preferred_element_type=f32)
    h = jnp.maximum(h + b0_ref[...], 0.0)
    h = jnp.maximum(jnp.dot(h, w1_ref[...], preferred_element_type=f32) + b1_ref[...], 0.0)
    h = jnp.maximum(jnp.dot(h, w2_ref[...], preferred_element_type=f32) + b2_ref[...], 0.0)
    mu = jnp.mean(h, axis=1, keepdims=True)
    var = jnp.mean((h - mu) * (h - mu), axis=1, keepdims=True)
    o_ref[...] = (h - mu) * lax.rsqrt(var + 1e-5) * g_ref[...] + bb_ref[...]


def _mlp_call(x, s, batch2d, u, W0x, W0e, W0u, b0, W1, b1, W2, b2, g, bb):
    grid = (N_NODES // BN,)
    full = lambda shape: pl.BlockSpec(shape, lambda i: (0, 0))
    return pl.pallas_call(
        _mlp_body,
        grid=grid,
        in_specs=[
            pl.BlockSpec((BN, D_FEAT), lambda i: (i, 0)),
            pl.BlockSpec((BN, D_EDGE), lambda i: (i, 0)),
            pl.BlockSpec((BN, 1), lambda i: (i, 0)),
            full((N_GRAPHS, D_GLOBAL)),
            full((D_FEAT, H1)),
            full((D_EDGE, H1)),
            full((D_GLOBAL, H1)),
            full((1, H1)),
            full((H1, H2)),
            full((1, H2)),
            full((H2, N_OUT)),
            full((1, N_OUT)),
            full((1, N_OUT)),
            full((1, N_OUT)),
        ],
        out_specs=pl.BlockSpec((BN, N_OUT), lambda i: (i, 0)),
        out_shape=jax.ShapeDtypeStruct((N_NODES, N_OUT), jnp.float32),
    )(x, s, batch2d, u, W0x, W0e, W0u, b0, W1, b1, W2, b2, g, bb)


def kernel(x, e, u, edge_index, batch, W0, b0, W1, b1, W2, b2, ln_scale, ln_bias):
    col = edge_index[1]
    s = jax.ops.segment_sum(e, col, num_segments=N_NODES)
    W0x = W0[:D_FEAT]
    W0e = W0[D_FEAT:D_FEAT + D_EDGE]
    W0u = W0[D_FEAT + D_EDGE:]
    return _mlp_call(
        x, s, batch[:, None], u, W0x, W0e, W0u, b0[None, :],
        W1, b1[None, :], W2, b2[None, :], ln_scale[None, :], ln_bias[None, :])


# speedup vs baseline: 4.7400x; 4.6926x over previous
import functools

import jax
import jax.numpy as jnp
from jax import lax
from jax.experimental import pallas as pl
from jax.experimental.pallas import tpu as pltpu
from jax.experimental.pallas import tpu_sc as plsc

N_NODES = 10000
N_EDGES = 320000
D_FEAT = 128
D_EDGE = 16
N_GRAPHS = 16
D_GLOBAL = 32
H1 = 256
H2 = 256
N_OUT = 128

BN = 1000

NC = 2
NS = 16
NW = NC * NS
EPR = 128
N_ROWS = N_EDGES // EPR
K_FULL = N_ROWS // NW
N_EXTRA = N_ROWS - K_FULL * NW
NPT = 1000


def _sc_scatter_build():
    mesh = plsc.VectorSubcoreMesh(core_axis_name="c", subcore_axis_name="s")

    @functools.partial(
        pl.kernel, mesh=mesh,
        compiler_params=pltpu.CompilerParams(use_tc_tiling_on_sc=False),
        out_type=jax.ShapeDtypeStruct((NC, N_NODES, D_EDGE), jnp.float32),
        scratch_types=[
            pltpu.VMEM((EPR,), jnp.int32),
            pltpu.VMEM((EPR,), jnp.int32),
            pltpu.VMEM((EPR, D_EDGE), jnp.float32),
            pltpu.VMEM((EPR, D_EDGE), jnp.float32),
            pltpu.VMEM((NPT, D_EDGE), jnp.float32),
            pltpu.VMEM_SHARED((N_NODES, D_EDGE), jnp.float32),
            pltpu.SemaphoreType.DMA,
            pltpu.SemaphoreType.DMA,
            pltpu.SemaphoreType.DMA,
            pltpu.SemaphoreType.DMA,
        ])
    def sc_scatter(col_hbm, e_hbm, out_hbm, idx0, idx1, ebuf0, ebuf1,
                   stage, acc, si0, si1, se0, se1):
        c = lax.axis_index("c")
        s = lax.axis_index("s")
        w = s * NC + c

        def zrow(i, carry):
            stage[i, :] = jnp.zeros((D_EDGE,), jnp.float32)
            return carry
        lax.fori_loop(0, NPT, zrow, 0)

        @pl.when(s < N_NODES // NPT)
        def _():
            pltpu.sync_copy(stage, acc.at[pl.ds(s * NPT, NPT)])
        plsc.subcore_barrier()

        def start(r, idxb, eb, semi, seme):
            pltpu.async_copy(col_hbm.at[r], idxb, semi)
            pltpu.async_copy(e_hbm.at[pl.ds(r * EPR, EPR)], eb, seme)

        def finish(idxb, eb, semi, seme):
            pltpu.make_async_copy(col_hbm.at[0], idxb, semi).wait()
            pltpu.make_async_copy(e_hbm.at[pl.ds(0, EPR)], eb, seme).wait()

        def scat(idxb, eb):
            pltpu.sync_copy(eb, acc.at[idxb], add=True)

        start(w, idx0, ebuf0, si0, se0)

        def body(g, carry):
            r1 = w + (2 * g + 1) * NW
            start(r1, idx1, ebuf1, si1, se1)
            finish(idx0, ebuf0, si0, se0)
            scat(idx0, ebuf0)

            @pl.when(g + 1 < K_FULL // 2)
            def _():
                start(w + (2 * g + 2) * NW, idx0, ebuf0, si0, se0)

            finish(idx1, ebuf1, si1, se1)
            scat(idx1, ebuf1)
            return carry

        lax.fori_loop(0, K_FULL // 2, body, 0)

        @pl.when(w < N_EXTRA)
        def _():
            r = w + K_FULL * NW
            start(r, idx0, ebuf0, si0, se0)
            finish(idx0, ebuf0, si0, se0)
            scat(idx0, ebuf0)

        plsc.subcore_barrier()

        @pl.when(s < N_NODES // NPT)
        def _():
            pltpu.sync_copy(acc.at[pl.ds(s * NPT, NPT)], stage)
            pltpu.sync_copy(stage, out_hbm.at[c, pl.ds(s * NPT, NPT)])

    return sc_scatter


_sc_scatter = _sc_scatter_build()


def _mlp_body(x_ref, s0_ref, s1_ref, bat_ref, u_ref, w0x_ref, w0e_ref, w0u_ref,
              b0_ref, w1_ref, b1_ref, w2_ref, b2_ref, g_ref, bb_ref, o_ref):
    f32 = jnp.float32
    uw = jnp.dot(u_ref[...], w0u_ref[...], preferred_element_type=f32)
    onehot = (bat_ref[...] == lax.broadcasted_iota(jnp.int32, (BN, N_GRAPHS), 1)
              ).astype(f32)
    h = jnp.dot(x_ref[...], w0x_ref[...], preferred_element_type=f32)
    h = h + jnp.dot(s0_ref[...] + s1_ref[...], w0e_ref[...], preferred_element_type=f32)
    h = h + jnp.dot(onehot, uw, preferred_element_type=f32)
    h = jnp.maximum(h + b0_ref[...], 0.0)
    h = jnp.maximum(jnp.dot(h, w1_ref[...], preferred_element_type=f32) + b1_ref[...], 0.0)
    h = jnp.maximum(jnp.dot(h, w2_ref[...], preferred_element_type=f32) + b2_ref[...], 0.0)
    mu = jnp.mean(h, axis=1, keepdims=True)
    var = jnp.mean((h - mu) * (h - mu), axis=1, keepdims=True)
    o_ref[...] = (h - mu) * lax.rsqrt(var + 1e-5) * g_ref[...] + bb_ref[...]


def _mlp_call(x, s0, s1, batch2d, u, W0x, W0e, W0u, b0, W1, b1, W2, b2, g, bb):
    grid = (N_NODES // BN,)
    full = lambda shape: pl.BlockSpec(shape, lambda i: (0, 0))
    return pl.pallas_call(
        _mlp_body,
        grid=grid,
        in_specs=[
            pl.BlockSpec((BN, D_FEAT), lambda i: (i, 0)),
            pl.BlockSpec((BN, D_EDGE), lambda i: (i, 0)),
            pl.BlockSpec((BN, D_EDGE), lambda i: (i, 0)),
            pl.BlockSpec((BN, 1), lambda i: (i, 0)),
            full((N_GRAPHS, D_GLOBAL)),
            full((D_FEAT, H1)),
            full((D_EDGE, H1)),
            full((D_GLOBAL, H1)),
            full((1, H1)),
            full((H1, H2)),
            full((1, H2)),
            full((H2, N_OUT)),
            full((1, N_OUT)),
            full((1, N_OUT)),
            full((1, N_OUT)),
        ],
        out_specs=pl.BlockSpec((BN, N_OUT), lambda i: (i, 0)),
        out_shape=jax.ShapeDtypeStruct((N_NODES, N_OUT), jnp.float32),
    )(x, s0, s1, batch2d, u, W0x, W0e, W0u, b0, W1, b1, W2, b2, g, bb)


def kernel(x, e, u, edge_index, batch, W0, b0, W1, b1, W2, b2, ln_scale, ln_bias):
    col2d = edge_index[1].reshape(N_ROWS, EPR)
    partial = _sc_scatter(col2d, e)
    W0x = W0[:D_FEAT]
    W0e = W0[D_FEAT:D_FEAT + D_EDGE]
    W0u = W0[D_FEAT + D_EDGE:]
    return _mlp_call(
        x, partial[0], partial[1], batch[:, None], u, W0x, W0e, W0u, b0[None, :],
        W1, b1[None, :], W2, b2[None, :], ln_scale[None, :], ln_bias[None, :])


# speedup vs baseline: 4.8950x; 1.0327x over previous
import functools

import jax
import jax.numpy as jnp
from jax import lax
from jax.experimental import pallas as pl
from jax.experimental.pallas import tpu as pltpu
from jax.experimental.pallas import tpu_sc as plsc

N_NODES = 10000
N_EDGES = 320000
D_FEAT = 128
D_EDGE = 16
N_GRAPHS = 16
D_GLOBAL = 32
H1 = 256
H2 = 256
N_OUT = 128

BN = 1000

NC = 2
NS = 16
NW = NC * NS
EPR = 128
N_ROWS = N_EDGES // EPR
K_FULL = N_ROWS // NW
N_EXTRA = N_ROWS - K_FULL * NW
NPT = 1000


def _sc_scatter_build():
    mesh = plsc.VectorSubcoreMesh(core_axis_name="c", subcore_axis_name="s")

    @functools.partial(
        pl.kernel, mesh=mesh,
        compiler_params=pltpu.CompilerParams(use_tc_tiling_on_sc=False),
        out_type=(jax.ShapeDtypeStruct((N_NODES, D_EDGE), jnp.float32),
                  jax.ShapeDtypeStruct((N_NODES, D_EDGE), jnp.float32)),
        scratch_types=[
            pltpu.VMEM((EPR,), jnp.int32),
            pltpu.VMEM((EPR,), jnp.int32),
            pltpu.VMEM((EPR, D_EDGE), jnp.float32),
            pltpu.VMEM((EPR, D_EDGE), jnp.float32),
            pltpu.VMEM((NPT, D_EDGE), jnp.float32),
            pltpu.VMEM_SHARED((N_NODES, D_EDGE), jnp.float32),
            pltpu.SemaphoreType.DMA,
            pltpu.SemaphoreType.DMA,
            pltpu.SemaphoreType.DMA,
            pltpu.SemaphoreType.DMA,
        ])
    def sc_scatter(col_hbm, e_hbm, out0_hbm, out1_hbm, idx0, idx1, ebuf0, ebuf1,
                   stage, acc, si0, si1, se0, se1):
        c = lax.axis_index("c")
        s = lax.axis_index("s")
        w = s * NC + c

        def zrow(i, carry):
            stage[i, :] = jnp.zeros((D_EDGE,), jnp.float32)
            return carry
        lax.fori_loop(0, NPT, zrow, 0)

        @pl.when(s < N_NODES // NPT)
        def _():
            pltpu.sync_copy(stage, acc.at[pl.ds(s * NPT, NPT)])
        plsc.subcore_barrier()

        def start(r, idxb, eb, semi, seme):
            pltpu.async_copy(col_hbm.at[pl.ds(r * EPR, EPR)], idxb, semi)
            pltpu.async_copy(e_hbm.at[pl.ds(r * EPR, EPR)], eb, seme)

        def finish(idxb, eb, semi, seme):
            pltpu.make_async_copy(col_hbm.at[pl.ds(0, EPR)], idxb, semi).wait()
            pltpu.make_async_copy(e_hbm.at[pl.ds(0, EPR)], eb, seme).wait()

        def scat(idxb, eb):
            pltpu.sync_copy(eb, acc.at[idxb], add=True)

        start(w, idx0, ebuf0, si0, se0)

        def body(g, carry):
            r1 = w + (2 * g + 1) * NW
            start(r1, idx1, ebuf1, si1, se1)
            finish(idx0, ebuf0, si0, se0)
            scat(idx0, ebuf0)

            @pl.when(g + 1 < K_FULL // 2)
            def _():
                start(w + (2 * g + 2) * NW, idx0, ebuf0, si0, se0)

            finish(idx1, ebuf1, si1, se1)
            scat(idx1, ebuf1)
            return carry

        lax.fori_loop(0, K_FULL // 2, body, 0)

        @pl.when(w < N_EXTRA)
        def _():
            r = w + K_FULL * NW
            start(r, idx0, ebuf0, si0, se0)
            finish(idx0, ebuf0, si0, se0)
            scat(idx0, ebuf0)

        plsc.subcore_barrier()

        @pl.when(s < N_NODES // NPT)
        def _():
            pltpu.sync_copy(acc.at[pl.ds(s * NPT, NPT)], stage)

            @pl.when(c == 0)
            def _():
                pltpu.sync_copy(stage, out0_hbm.at[pl.ds(s * NPT, NPT)])

            @pl.when(c == 1)
            def _():
                pltpu.sync_copy(stage, out1_hbm.at[pl.ds(s * NPT, NPT)])

    return sc_scatter


_sc_scatter = _sc_scatter_build()


def _mlp_body(x_ref, s0_ref, s1_ref, bat_ref, u_ref, w0x_ref, w0e_ref, w0u_ref,
              b0_ref, w1_ref, b1_ref, w2_ref, b2_ref, g_ref, bb_ref, o_ref):
    f32 = jnp.float32
    uw = jnp.dot(u_ref[...], w0u_ref[...], preferred_element_type=f32)
    onehot = (bat_ref[...] == lax.broadcasted_iota(jnp.int32, (BN, N_GRAPHS), 1)
              ).astype(f32)
    h = jnp.dot(x_ref[...], w0x_ref[...], preferred_element_type=f32)
    h = h + jnp.dot(s0_ref[...] + s1_ref[...], w0e_ref[...], preferred_element_type=f32)
    h = h + jnp.dot(onehot, uw, preferred_element_type=f32)
    h = jnp.maximum(h + b0_ref[...], 0.0)
    h = jnp.maximum(jnp.dot(h, w1_ref[...], preferred_element_type=f32) + b1_ref[...], 0.0)
    h = jnp.maximum(jnp.dot(h, w2_ref[...], preferred_element_type=f32) + b2_ref[...], 0.0)
    mu = jnp.mean(h, axis=1, keepdims=True)
    var = jnp.mean((h - mu) * (h - mu), axis=1, keepdims=True)
    o_ref[...] = (h - mu) * lax.rsqrt(var + 1e-5) * g_ref[...] + bb_ref[...]


def _mlp_call(x, s0, s1, batch2d, u, W0x, W0e, W0u, b0, W1, b1, W2, b2, g, bb):
    grid = (N_NODES // BN,)
    full = lambda shape: pl.BlockSpec(shape, lambda i: (0, 0))
    return pl.pallas_call(
        _mlp_body,
        grid=grid,
        in_specs=[
            pl.BlockSpec((BN, D_FEAT), lambda i: (i, 0)),
            pl.BlockSpec((BN, D_EDGE), lambda i: (i, 0)),
            pl.BlockSpec((BN, D_EDGE), lambda i: (i, 0)),
            pl.BlockSpec((BN, 1), lambda i: (i, 0)),
            full((N_GRAPHS, D_GLOBAL)),
            full((D_FEAT, H1)),
            full((D_EDGE, H1)),
            full((D_GLOBAL, H1)),
            full((1, H1)),
            full((H1, H2)),
            full((1, H2)),
            full((H2, N_OUT)),
            full((1, N_OUT)),
            full((1, N_OUT)),
            full((1, N_OUT)),
        ],
        out_specs=pl.BlockSpec((BN, N_OUT), lambda i: (i, 0)),
        out_shape=jax.ShapeDtypeStruct((N_NODES, N_OUT), jnp.float32),
    )(x, s0, s1, batch2d, u, W0x, W0e, W0u, b0, W1, b1, W2, b2, g, bb)


def kernel(x, e, u, edge_index, batch, W0, b0, W1, b1, W2, b2, ln_scale, ln_bias):
    s0, s1 = _sc_scatter(edge_index[1], e)
    W0x = W0[:D_FEAT]
    W0e = W0[D_FEAT:D_FEAT + D_EDGE]
    W0u = W0[D_FEAT + D_EDGE:]
    return _mlp_call(
        x, s0, s1, batch[:, None], u, W0x, W0e, W0u, b0[None, :],
        W1, b1[None, :], W2, b2[None, :], ln_scale[None, :], ln_bias[None, :])


# speedup vs baseline: 4.9412x; 1.0095x over previous
import functools

import jax
import jax.numpy as jnp
from jax import lax
from jax.experimental import pallas as pl
from jax.experimental.pallas import tpu as pltpu
from jax.experimental.pallas import tpu_sc as plsc

N_NODES = 10000
N_EDGES = 320000
D_FEAT = 128
D_EDGE = 16
N_GRAPHS = 16
D_GLOBAL = 32
H1 = 256
H2 = 256
N_OUT = 128

BN = 1000

NC = 2
NS = 16
NW = NC * NS
EPR = 128
N_ROWS = N_EDGES // EPR
K_FULL = N_ROWS // NW
N_EXTRA = N_ROWS - K_FULL * NW
NPT = 1000


def _sc_scatter_build():
    mesh = plsc.VectorSubcoreMesh(core_axis_name="c", subcore_axis_name="s")
    NBUF = 6

    @functools.partial(
        pl.kernel, mesh=mesh,
        compiler_params=pltpu.CompilerParams(
            use_tc_tiling_on_sc=False, needs_layout_passes=False),
        out_type=(jax.ShapeDtypeStruct((N_NODES, D_EDGE), jnp.float32),
                  jax.ShapeDtypeStruct((N_NODES, D_EDGE), jnp.float32)),
        scratch_types=(
            [pltpu.VMEM((EPR,), jnp.int32) for _ in range(NBUF)]
            + [pltpu.VMEM((D_EDGE, EPR), jnp.float32) for _ in range(NBUF)]
            + [pltpu.VMEM((EPR, D_EDGE), jnp.float32) for _ in range(NBUF)]
            + [pltpu.VMEM((NPT, D_EDGE), jnp.float32),
               pltpu.VMEM_SHARED((N_NODES, D_EDGE), jnp.float32)]
            + [pltpu.SemaphoreType.DMA for _ in range(2 * NBUF)]
        ))
    def sc_scatter(eT_hbm, col_hbm, out0_hbm, out1_hbm, *bufs):
        idx = bufs[0:NBUF]
        tb = bufs[NBUF:2 * NBUF]
        sb = bufs[2 * NBUF:3 * NBUF]
        stage = bufs[3 * NBUF]
        acc = bufs[3 * NBUF + 1]
        sl = bufs[3 * NBUF + 2:3 * NBUF + 2 + NBUF]
        ss = bufs[3 * NBUF + 2 + NBUF:3 * NBUF + 2 + 2 * NBUF]

        c = lax.axis_index("c")
        s = lax.axis_index("s")
        w = s * NC + c
        iota16 = lax.broadcasted_iota(jnp.int32, (D_EDGE,), 0)

        def zrow(i, carry):
            stage[i, :] = jnp.zeros((D_EDGE,), jnp.float32)
            return carry
        lax.fori_loop(0, NPT, zrow, 0)

        @pl.when(s < N_NODES // NPT)
        def _():
            pltpu.sync_copy(stage, acc.at[pl.ds(s * NPT, NPT)])
        plsc.subcore_barrier()

        def start_load(k, b):
            r = w + k * NW
            pltpu.async_copy(col_hbm.at[pl.ds(r * EPR, EPR)], idx[b], sl[b])
            pltpu.async_copy(eT_hbm.at[:, pl.ds(r * EPR, EPR)], tb[b], sl[b])

        def wait_load(b):
            pltpu.make_async_copy(col_hbm.at[pl.ds(0, EPR)], idx[b], sl[b]).wait()
            pltpu.make_async_copy(eT_hbm.at[:, pl.ds(0, EPR)], tb[b], sl[b]).wait()

        def transpose(b):
            UN = 8

            def tr(j0, carry):
                for u in range(UN):
                    j = j0 * UN + u
                    v = plsc.load_gather(tb[b], [iota16, jnp.full((D_EDGE,), j, jnp.int32)])
                    sb[b][j, :] = v
                return carry
            lax.fori_loop(0, EPR // UN, tr, 0)

        def start_scat(b):
            pltpu.async_copy(sb[b], acc.at[idx[b]], ss[b], add=True)

        def wait_scat(b):
            pltpu.make_async_copy(sb[b], acc.at[idx[b]], ss[b]).wait()

        for b in range(3):
            start_load(b, b)

        def body(g, carry):
            for i in range(NBUF):
                b = i
                k = g * NBUF + i

                wait_load(b)
                transpose(b)
                start_scat(b)

                bb = (i + 3) % NBUF

                @pl.when(k + 3 < K_FULL)
                def _():
                    @pl.when(k >= 3)
                    def _():
                        wait_scat(bb)
                    start_load(k + 3, bb)
            return carry

        lax.fori_loop(0, K_FULL // NBUF, body, 0)

        @pl.when(w < N_EXTRA)
        def _():
            wait_scat(0)
            start_load(K_FULL, 0)
            wait_load(0)
            transpose(0)
            start_scat(0)

        for b in range(NBUF):
            wait_scat(b)

        plsc.subcore_barrier()

        @pl.when(s < N_NODES // NPT)
        def _():
            pltpu.sync_copy(acc.at[pl.ds(s * NPT, NPT)], stage)

            @pl.when(c == 0)
            def _():
                pltpu.sync_copy(stage, out0_hbm.at[pl.ds(s * NPT, NPT)])

            @pl.when(c == 1)
            def _():
                pltpu.sync_copy(stage, out1_hbm.at[pl.ds(s * NPT, NPT)])

    return sc_scatter


_sc_scatter = _sc_scatter_build()


def _mlp_body(x_ref, s0_ref, s1_ref, bat_ref, u_ref, w0x_ref, w0e_ref, w0u_ref,
              b0_ref, w1_ref, b1_ref, w2_ref, b2_ref, g_ref, bb_ref, o_ref):
    f32 = jnp.float32
    uw = jnp.dot(u_ref[...], w0u_ref[...], preferred_element_type=f32)
    onehot = (bat_ref[...] == lax.broadcasted_iota(jnp.int32, (BN, N_GRAPHS), 1)
              ).astype(f32)
    h = jnp.dot(x_ref[...], w0x_ref[...], preferred_element_type=f32)
    h = h + jnp.dot(s0_ref[...] + s1_ref[...], w0e_ref[...], preferred_element_type=f32)
    h = h + jnp.dot(onehot, uw, preferred_element_type=f32)
    h = jnp.maximum(h + b0_ref[...], 0.0)
    h = jnp.maximum(jnp.dot(h, w1_ref[...], preferred_element_type=f32) + b1_ref[...], 0.0)
    h = jnp.maximum(jnp.dot(h, w2_ref[...], preferred_element_type=f32) + b2_ref[...], 0.0)
    mu = jnp.mean(h, axis=1, keepdims=True)
    var = jnp.mean((h - mu) * (h - mu), axis=1, keepdims=True)
    o_ref[...] = (h - mu) * lax.rsqrt(var + 1e-5) * g_ref[...] + bb_ref[...]


def _mlp_call(x, s0, s1, batch2d, u, W0x, W0e, W0u, b0, W1, b1, W2, b2, g, bb):
    grid = (N_NODES // BN,)
    full = lambda shape: pl.BlockSpec(shape, lambda i: (0, 0))
    return pl.pallas_call(
        _mlp_body,
        grid=grid,
        in_specs=[
            pl.BlockSpec((BN, D_FEAT), lambda i: (i, 0)),
            pl.BlockSpec((BN, D_EDGE), lambda i: (i, 0)),
            pl.BlockSpec((BN, D_EDGE), lambda i: (i, 0)),
            pl.BlockSpec((BN, 1), lambda i: (i, 0)),
            full((N_GRAPHS, D_GLOBAL)),
            full((D_FEAT, H1)),
            full((D_EDGE, H1)),
            full((D_GLOBAL, H1)),
            full((1, H1)),
            full((H1, H2)),
            full((1, H2)),
            full((H2, N_OUT)),
            full((1, N_OUT)),
            full((1, N_OUT)),
            full((1, N_OUT)),
        ],
        out_specs=pl.BlockSpec((BN, N_OUT), lambda i: (i, 0)),
        out_shape=jax.ShapeDtypeStruct((N_NODES, N_OUT), jnp.float32),
    )(x, s0, s1, batch2d, u, W0x, W0e, W0u, b0, W1, b1, W2, b2, g, bb)


def kernel(x, e, u, edge_index, batch, W0, b0, W1, b1, W2, b2, ln_scale, ln_bias):
    s0, s1 = _sc_scatter(e.T, edge_index[1])
    W0x = W0[:D_FEAT]
    W0e = W0[D_FEAT:D_FEAT + D_EDGE]
    W0u = W0[D_FEAT + D_EDGE:]
    return _mlp_call(
        x, s0, s1, batch[:, None], u, W0x, W0e, W0u, b0[None, :],
        W1, b1[None, :], W2, b2[None, :], ln_scale[None, :], ln_bias[None, :])


# speedup vs baseline: 8.1673x; 1.6529x over previous
import functools

import jax
import jax.numpy as jnp
from jax import lax
from jax.experimental import pallas as pl
from jax.experimental.pallas import tpu as pltpu
from jax.experimental.pallas import tpu_sc as plsc

N_NODES = 10000
N_EDGES = 320000
D_FEAT = 128
D_EDGE = 16
N_GRAPHS = 16
D_GLOBAL = 32
H1 = 256
H2 = 256
N_OUT = 128

BN = 1000

NC = 2
NS = 16
NW = NC * NS
EPR = 128
N_ROWS = N_EDGES // EPR
K_FULL = N_ROWS // NW
N_EXTRA = N_ROWS - K_FULL * NW
NPT = 1000


def _sc_scatter_build():
    mesh = plsc.VectorSubcoreMesh(core_axis_name="c", subcore_axis_name="s")
    NBUF = 6

    @functools.partial(
        pl.kernel, mesh=mesh,
        compiler_params=pltpu.CompilerParams(
            use_tc_tiling_on_sc=False, needs_layout_passes=False),
        out_type=(jax.ShapeDtypeStruct((N_NODES, D_EDGE), jnp.float32),
                  jax.ShapeDtypeStruct((N_NODES, D_EDGE), jnp.float32)),
        scratch_types=(
            [pltpu.VMEM((EPR,), jnp.int32) for _ in range(NBUF)]
            + [pltpu.VMEM((D_EDGE, EPR), jnp.float32) for _ in range(NBUF)]
            + [pltpu.VMEM((EPR, D_EDGE), jnp.float32) for _ in range(NBUF)]
            + [pltpu.VMEM((NPT, D_EDGE), jnp.float32),
               pltpu.VMEM_SHARED((N_NODES, D_EDGE), jnp.float32)]
            + [pltpu.SemaphoreType.DMA for _ in range(2 * NBUF)]
        ))
    def sc_scatter(eT_hbm, col_hbm, out0_hbm, out1_hbm, *bufs):
        idx = bufs[0:NBUF]
        tb = bufs[NBUF:2 * NBUF]
        sb = bufs[2 * NBUF:3 * NBUF]
        stage = bufs[3 * NBUF]
        acc = bufs[3 * NBUF + 1]
        sl = bufs[3 * NBUF + 2:3 * NBUF + 2 + NBUF]
        ss = bufs[3 * NBUF + 2 + NBUF:3 * NBUF + 2 + 2 * NBUF]

        c = lax.axis_index("c")
        s = lax.axis_index("s")
        w = s * NC + c
        iota16 = lax.broadcasted_iota(jnp.int32, (D_EDGE,), 0)

        def zrow(i, carry):
            stage[i, :] = jnp.zeros((D_EDGE,), jnp.float32)
            return carry
        lax.fori_loop(0, NPT, zrow, 0)

        @pl.when(s < N_NODES // NPT)
        def _():
            pltpu.sync_copy(stage, acc.at[pl.ds(s * NPT, NPT)])
        plsc.subcore_barrier()

        def start_load(k, b):
            r = w + k * NW
            pltpu.async_copy(col_hbm.at[pl.ds(r * EPR, EPR)], idx[b], sl[b])
            pltpu.async_copy(eT_hbm.at[:, pl.ds(r * EPR, EPR)], tb[b], sl[b])

        def wait_load(b):
            pltpu.make_async_copy(col_hbm.at[pl.ds(0, EPR)], idx[b], sl[b]).wait()
            pltpu.make_async_copy(eT_hbm.at[:, pl.ds(0, EPR)], tb[b], sl[b]).wait()

        rot = [((iota16 + d) & 15) for d in range(D_EDGE)]

        def transpose(b):
            def tr(m, carry):
                base = m * D_EDGE
                for d in range(D_EDGE):
                    ecol = rot[d] + base
                    v = plsc.load_gather(tb[b], [iota16, ecol])
                    plsc.store_scatter(sb[b], [ecol, iota16], v)
                return carry
            lax.fori_loop(0, EPR // D_EDGE, tr, 0)

        def start_scat(b):
            pltpu.async_copy(sb[b], acc.at[idx[b]], ss[b], add=True)

        def wait_scat(b):
            pltpu.make_async_copy(sb[b], acc.at[idx[b]], ss[b]).wait()

        for b in range(3):
            start_load(b, b)

        def body(g, carry):
            for i in range(NBUF):
                b = i
                k = g * NBUF + i

                wait_load(b)
                transpose(b)
                start_scat(b)

                bb = (i + 3) % NBUF

                @pl.when(k + 3 < K_FULL)
                def _():
                    @pl.when(k >= 3)
                    def _():
                        wait_scat(bb)
                    start_load(k + 3, bb)
            return carry

        lax.fori_loop(0, K_FULL // NBUF, body, 0)

        @pl.when(w < N_EXTRA)
        def _():
            wait_scat(0)
            start_load(K_FULL, 0)
            wait_load(0)
            transpose(0)
            start_scat(0)

        for b in range(NBUF):
            wait_scat(b)

        plsc.subcore_barrier()

        @pl.when(s < N_NODES // NPT)
        def _():
            pltpu.sync_copy(acc.at[pl.ds(s * NPT, NPT)], stage)

            @pl.when(c == 0)
            def _():
                pltpu.sync_copy(stage, out0_hbm.at[pl.ds(s * NPT, NPT)])

            @pl.when(c == 1)
            def _():
                pltpu.sync_copy(stage, out1_hbm.at[pl.ds(s * NPT, NPT)])

    return sc_scatter


_sc_scatter = _sc_scatter_build()


def _mlp_body(x_ref, s0_ref, s1_ref, bat_ref, u_ref, w0x_ref, w0e_ref, w0u_ref,
              b0_ref, w1_ref, b1_ref, w2_ref, b2_ref, g_ref, bb_ref, o_ref):
    f32 = jnp.float32
    uw = jnp.dot(u_ref[...], w0u_ref[...], preferred_element_type=f32)
    onehot = (bat_ref[...] == lax.broadcasted_iota(jnp.int32, (BN, N_GRAPHS), 1)
              ).astype(f32)
    h = jnp.dot(x_ref[...], w0x_ref[...], preferred_element_type=f32)
    h = h + jnp.dot(s0_ref[...] + s1_ref[...], w0e_ref[...], preferred_element_type=f32)
    h = h + jnp.dot(onehot, uw, preferred_element_type=f32)
    h = jnp.maximum(h + b0_ref[...], 0.0)
    h = jnp.maximum(jnp.dot(h, w1_ref[...], preferred_element_type=f32) + b1_ref[...], 0.0)
    h = jnp.maximum(jnp.dot(h, w2_ref[...], preferred_element_type=f32) + b2_ref[...], 0.0)
    mu = jnp.mean(h, axis=1, keepdims=True)
    var = jnp.mean((h - mu) * (h - mu), axis=1, keepdims=True)
    o_ref[...] = (h - mu) * lax.rsqrt(var + 1e-5) * g_ref[...] + bb_ref[...]


def _mlp_call(x, s0, s1, batch2d, u, W0x, W0e, W0u, b0, W1, b1, W2, b2, g, bb):
    grid = (N_NODES // BN,)
    full = lambda shape: pl.BlockSpec(shape, lambda i: (0, 0))
    return pl.pallas_call(
        _mlp_body,
        grid=grid,
        in_specs=[
            pl.BlockSpec((BN, D_FEAT), lambda i: (i, 0)),
            pl.BlockSpec((BN, D_EDGE), lambda i: (i, 0)),
            pl.BlockSpec((BN, D_EDGE), lambda i: (i, 0)),
            pl.BlockSpec((BN, 1), lambda i: (i, 0)),
            full((N_GRAPHS, D_GLOBAL)),
            full((D_FEAT, H1)),
            full((D_EDGE, H1)),
            full((D_GLOBAL, H1)),
            full((1, H1)),
            full((H1, H2)),
            full((1, H2)),
            full((H2, N_OUT)),
            full((1, N_OUT)),
            full((1, N_OUT)),
            full((1, N_OUT)),
        ],
        out_specs=pl.BlockSpec((BN, N_OUT), lambda i: (i, 0)),
        out_shape=jax.ShapeDtypeStruct((N_NODES, N_OUT), jnp.float32),
    )(x, s0, s1, batch2d, u, W0x, W0e, W0u, b0, W1, b1, W2, b2, g, bb)


def kernel(x, e, u, edge_index, batch, W0, b0, W1, b1, W2, b2, ln_scale, ln_bias):
    s0, s1 = _sc_scatter(e.T, edge_index[1])
    W0x = W0[:D_FEAT]
    W0e = W0[D_FEAT:D_FEAT + D_EDGE]
    W0u = W0[D_FEAT + D_EDGE:]
    return _mlp_call(
        x, s0, s1, batch[:, None], u, W0x, W0e, W0u, b0[None, :],
        W1, b1[None, :], W2, b2[None, :], ln_scale[None, :], ln_bias[None, :])


# speedup vs baseline: 10.4304x; 1.2771x over previous
import functools

import jax
import jax.numpy as jnp
from jax import lax
from jax.experimental import pallas as pl
from jax.experimental.pallas import tpu as pltpu
from jax.experimental.pallas import tpu_sc as plsc

N_NODES = 10000
N_EDGES = 320000
D_FEAT = 128
D_EDGE = 16
N_GRAPHS = 16
D_GLOBAL = 32
H1 = 256
H2 = 256
N_OUT = 128

BN = 1000

NC = 2
NS = 16
NW = NC * NS
EPR = 128
N_ROWS = N_EDGES // EPR
K_FULL = N_ROWS // NW
N_EXTRA = N_ROWS - K_FULL * NW
NPT = 1000


def _sc_scatter_build():
    mesh = plsc.VectorSubcoreMesh(core_axis_name="c", subcore_axis_name="s")
    NBUF = 6

    @functools.partial(
        pl.kernel, mesh=mesh,
        compiler_params=pltpu.CompilerParams(
            use_tc_tiling_on_sc=False, needs_layout_passes=False),
        out_type=(jax.ShapeDtypeStruct((N_NODES, D_EDGE), jnp.float32),
                  jax.ShapeDtypeStruct((N_NODES, D_EDGE), jnp.float32)),
        scratch_types=(
            [pltpu.VMEM((EPR,), jnp.int32) for _ in range(NBUF)]
            + [pltpu.VMEM((2, 8, EPR), jnp.float32) for _ in range(NBUF)]
            + [pltpu.VMEM((EPR, D_EDGE), jnp.float32) for _ in range(NBUF)]
            + [pltpu.VMEM((NPT, D_EDGE), jnp.float32),
               pltpu.VMEM_SHARED((N_NODES, D_EDGE), jnp.float32)]
            + [pltpu.SemaphoreType.DMA for _ in range(2 * NBUF)]
        ))
    def sc_scatter(ev_hbm, ei_hbm, out0_hbm, out1_hbm, *bufs):
        idx = bufs[0:NBUF]
        tb = bufs[NBUF:2 * NBUF]
        sb = bufs[2 * NBUF:3 * NBUF]
        stage = bufs[3 * NBUF]
        acc = bufs[3 * NBUF + 1]
        sl = bufs[3 * NBUF + 2:3 * NBUF + 2 + NBUF]
        ss = bufs[3 * NBUF + 2 + NBUF:3 * NBUF + 2 + 2 * NBUF]

        c = lax.axis_index("c")
        s = lax.axis_index("s")
        w = s * NC + c
        iota16 = lax.broadcasted_iota(jnp.int32, (D_EDGE,), 0)

        def zrow(i, carry):
            stage[i, :] = jnp.zeros((D_EDGE,), jnp.float32)
            return carry
        lax.fori_loop(0, NPT, zrow, 0)

        @pl.when(s < N_NODES // NPT)
        def _():
            pltpu.sync_copy(stage, acc.at[pl.ds(s * NPT, NPT)])
        plsc.subcore_barrier()

        def start_load(k, b):
            r = w + k * NW
            pltpu.async_copy(ei_hbm.at[r, 1], idx[b], sl[b])
            pltpu.async_copy(ev_hbm.at[:, r], tb[b], sl[b])

        def wait_load(b):
            pltpu.make_async_copy(ei_hbm.at[0, 1], idx[b], sl[b]).wait()
            pltpu.make_async_copy(ev_hbm.at[:, 0], tb[b], sl[b]).wait()

        rot = [((iota16 + d) & 15) for d in range(D_EDGE)]
        hi3 = iota16 >> 3
        lo3 = iota16 & 7

        def transpose(b):
            def tr(m, carry):
                base = m * D_EDGE
                for d in range(D_EDGE):
                    ecol = rot[d] + base
                    v = plsc.load_gather(tb[b], [hi3, lo3, ecol])
                    plsc.store_scatter(sb[b], [ecol, iota16], v)
                return carry
            lax.fori_loop(0, EPR // D_EDGE, tr, 0)

        def start_scat(b):
            pltpu.async_copy(sb[b], acc.at[idx[b]], ss[b], add=True)

        def wait_scat(b):
            pltpu.make_async_copy(sb[b], acc.at[idx[b]], ss[b]).wait()

        for b in range(3):
            start_load(b, b)

        def body(g, carry):
            for i in range(NBUF):
                b = i
                k = g * NBUF + i

                wait_load(b)
                transpose(b)
                start_scat(b)

                bb = (i + 3) % NBUF

                @pl.when(k + 3 < K_FULL)
                def _():
                    @pl.when(k >= 3)
                    def _():
                        wait_scat(bb)
                    start_load(k + 3, bb)
            return carry

        lax.fori_loop(0, K_FULL // NBUF, body, 0)

        @pl.when(w < N_EXTRA)
        def _():
            wait_scat(0)
            start_load(K_FULL, 0)
            wait_load(0)
            transpose(0)
            start_scat(0)

        for b in range(NBUF):
            wait_scat(b)

        plsc.subcore_barrier()

        @pl.when(s < N_NODES // NPT)
        def _():
            pltpu.sync_copy(acc.at[pl.ds(s * NPT, NPT)], stage)

            @pl.when(c == 0)
            def _():
                pltpu.sync_copy(stage, out0_hbm.at[pl.ds(s * NPT, NPT)])

            @pl.when(c == 1)
            def _():
                pltpu.sync_copy(stage, out1_hbm.at[pl.ds(s * NPT, NPT)])

    return sc_scatter


_sc_scatter = _sc_scatter_build()


def _mlp_body(x_ref, s0_ref, s1_ref, bat_ref, u_ref, w0x_ref, w0e_ref, w0u_ref,
              b0_ref, w1_ref, b1_ref, w2_ref, b2_ref, g_ref, bb_ref, o_ref):
    f32 = jnp.float32
    uw = jnp.dot(u_ref[...], w0u_ref[...], preferred_element_type=f32)
    onehot = (bat_ref[...] == lax.broadcasted_iota(jnp.int32, (BN, N_GRAPHS), 1)
              ).astype(f32)
    h = jnp.dot(x_ref[...], w0x_ref[...], preferred_element_type=f32)
    h = h + jnp.dot(s0_ref[...] + s1_ref[...], w0e_ref[...], preferred_element_type=f32)
    h = h + jnp.dot(onehot, uw, preferred_element_type=f32)
    h = jnp.maximum(h + b0_ref[...], 0.0)
    h = jnp.maximum(jnp.dot(h, w1_ref[...], preferred_element_type=f32) + b1_ref[...], 0.0)
    h = jnp.maximum(jnp.dot(h, w2_ref[...], preferred_element_type=f32) + b2_ref[...], 0.0)
    mu = jnp.mean(h, axis=1, keepdims=True)
    var = jnp.mean((h - mu) * (h - mu), axis=1, keepdims=True)
    o_ref[...] = (h - mu) * lax.rsqrt(var + 1e-5) * g_ref[...] + bb_ref[...]


def _mlp_call(x, s0, s1, batch2d, u, W0x, W0e, W0u, b0, W1, b1, W2, b2, g, bb):
    grid = (N_NODES // BN,)
    full = lambda shape: pl.BlockSpec(shape, lambda i: (0, 0))
    return pl.pallas_call(
        _mlp_body,
        grid=grid,
        in_specs=[
            pl.BlockSpec((BN, D_FEAT), lambda i: (i, 0)),
            pl.BlockSpec((BN, D_EDGE), lambda i: (i, 0)),
            pl.BlockSpec((BN, D_EDGE), lambda i: (i, 0)),
            pl.BlockSpec((BN, 1), lambda i: (i, 0)),
            full((N_GRAPHS, D_GLOBAL)),
            full((D_FEAT, H1)),
            full((D_EDGE, H1)),
            full((D_GLOBAL, H1)),
            full((1, H1)),
            full((H1, H2)),
            full((1, H2)),
            full((H2, N_OUT)),
            full((1, N_OUT)),
            full((1, N_OUT)),
            full((1, N_OUT)),
        ],
        out_specs=pl.BlockSpec((BN, N_OUT), lambda i: (i, 0)),
        out_shape=jax.ShapeDtypeStruct((N_NODES, N_OUT), jnp.float32),
    )(x, s0, s1, batch2d, u, W0x, W0e, W0u, b0, W1, b1, W2, b2, g, bb)


def kernel(x, e, u, edge_index, batch, W0, b0, W1, b1, W2, b2, ln_scale, ln_bias):
    ev = e.T.reshape(2, 8, N_ROWS, EPR).transpose(0, 2, 1, 3)
    ei = edge_index.reshape(2, N_ROWS, EPR).transpose(1, 0, 2)
    s0, s1 = _sc_scatter(ev, ei)
    W0x = W0[:D_FEAT]
    W0e = W0[D_FEAT:D_FEAT + D_EDGE]
    W0u = W0[D_FEAT + D_EDGE:]
    return _mlp_call(
        x, s0, s1, batch[:, None], u, W0x, W0e, W0u, b0[None, :],
        W1, b1[None, :], W2, b2[None, :], ln_scale[None, :], ln_bias[None, :])


# speedup vs baseline: 11.0879x; 1.0630x over previous
import functools

import jax
import jax.numpy as jnp
from jax import lax
from jax.experimental import pallas as pl
from jax.experimental.pallas import tpu as pltpu
from jax.experimental.pallas import tpu_sc as plsc

N_NODES = 10000
N_EDGES = 320000
D_FEAT = 128
D_EDGE = 16
N_GRAPHS = 16
D_GLOBAL = 32
H1 = 256
H2 = 256
N_OUT = 128

BN = 1000

NC = 2
NS = 16
NW = NC * NS
EPR = 128
N_ROWS = N_EDGES // EPR
K_FULL = N_ROWS // NW
N_EXTRA = N_ROWS - K_FULL * NW
NPT = 1000


def _sc_scatter_build():
    mesh = plsc.VectorSubcoreMesh(core_axis_name="c", subcore_axis_name="s")
    NBUF = 6

    @functools.partial(
        pl.kernel, mesh=mesh,
        compiler_params=pltpu.CompilerParams(
            use_tc_tiling_on_sc=False, needs_layout_passes=False),
        out_type=(jax.ShapeDtypeStruct((N_NODES, 128), jnp.float32),
                  jax.ShapeDtypeStruct((N_NODES, 128), jnp.float32)),
        scratch_types=(
            [pltpu.VMEM((EPR,), jnp.int32) for _ in range(NBUF)]
            + [pltpu.VMEM((2, 8, EPR), jnp.float32) for _ in range(NBUF)]
            + [pltpu.VMEM((EPR, D_EDGE), jnp.float32) for _ in range(NBUF)]
            + [pltpu.VMEM((NPT, D_EDGE), jnp.float32),
               pltpu.VMEM_SHARED((N_NODES, D_EDGE), jnp.float32)]
            + [pltpu.SemaphoreType.DMA for _ in range(2 * NBUF)]
        ))
    def sc_scatter(ev_hbm, ei_hbm, out0_hbm, out1_hbm, *bufs):
        idx = bufs[0:NBUF]
        tb = bufs[NBUF:2 * NBUF]
        sb = bufs[2 * NBUF:3 * NBUF]
        stage = bufs[3 * NBUF]
        acc = bufs[3 * NBUF + 1]
        sl = bufs[3 * NBUF + 2:3 * NBUF + 2 + NBUF]
        ss = bufs[3 * NBUF + 2 + NBUF:3 * NBUF + 2 + 2 * NBUF]

        c = lax.axis_index("c")
        s = lax.axis_index("s")
        w = s * NC + c
        iota16 = lax.broadcasted_iota(jnp.int32, (D_EDGE,), 0)

        def zrow(i, carry):
            stage[i, :] = jnp.zeros((D_EDGE,), jnp.float32)
            return carry
        lax.fori_loop(0, NPT, zrow, 0)

        @pl.when(s < N_NODES // NPT)
        def _():
            pltpu.sync_copy(stage, acc.at[pl.ds(s * NPT, NPT)])
        plsc.subcore_barrier()

        def start_load(k, b):
            r = w + k * NW
            pltpu.async_copy(ei_hbm.at[r, 1], idx[b], sl[b])
            pltpu.async_copy(ev_hbm.at[:, r], tb[b], sl[b])

        def wait_load(b):
            pltpu.make_async_copy(ei_hbm.at[0, 1], idx[b], sl[b]).wait()
            pltpu.make_async_copy(ev_hbm.at[:, 0], tb[b], sl[b]).wait()

        rot = [((iota16 + d) & 15) for d in range(D_EDGE)]
        hi3 = iota16 >> 3
        lo3 = iota16 & 7

        def transpose(b):
            def tr(m, carry):
                base = m * D_EDGE
                for d in range(D_EDGE):
                    ecol = rot[d] + base
                    v = plsc.load_gather(tb[b], [hi3, lo3, ecol])
                    plsc.store_scatter(sb[b], [ecol, iota16], v)
                return carry
            lax.fori_loop(0, EPR // D_EDGE, tr, 0)

        def start_scat(b):
            pltpu.async_copy(sb[b], acc.at[idx[b]], ss[b], add=True)

        def wait_scat(b):
            pltpu.make_async_copy(sb[b], acc.at[idx[b]], ss[b]).wait()

        for b in range(3):
            start_load(b, b)

        def body(g, carry):
            for i in range(NBUF):
                b = i
                k = g * NBUF + i

                wait_load(b)
                transpose(b)
                start_scat(b)

                bb = (i + 3) % NBUF

                @pl.when(k + 3 < K_FULL)
                def _():
                    @pl.when(k >= 3)
                    def _():
                        wait_scat(bb)
                    start_load(k + 3, bb)
            return carry

        lax.fori_loop(0, K_FULL // NBUF, body, 0)

        @pl.when(w < N_EXTRA)
        def _():
            wait_scat(0)
            start_load(K_FULL, 0)
            wait_load(0)
            transpose(0)
            start_scat(0)

        for b in range(NBUF):
            wait_scat(b)

        plsc.subcore_barrier()

        @pl.when(s < N_NODES // NPT)
        def _():
            src_slice = acc.at[pl.ds(s * NPT, NPT)]

            @pl.when(c == 0)
            def _():
                pltpu.sync_copy(src_slice,
                                out0_hbm.at[pl.ds(s * NPT, NPT), pl.ds(0, D_EDGE)])

            @pl.when(c == 1)
            def _():
                pltpu.sync_copy(src_slice,
                                out1_hbm.at[pl.ds(s * NPT, NPT), pl.ds(0, D_EDGE)])

    return sc_scatter


_sc_scatter = _sc_scatter_build()


def _mlp_body(x_ref, s0_ref, s1_ref, bat_ref, u_ref, w0x_ref, w0e_ref, w0u_ref,
              b0_ref, w1_ref, b1_ref, w2_ref, b2_ref, g_ref, bb_ref, o_ref):
    f32 = jnp.float32
    uw = jnp.dot(u_ref[...], w0u_ref[...], preferred_element_type=f32)
    onehot = (bat_ref[...] == lax.broadcasted_iota(jnp.int32, (BN, N_GRAPHS), 1)
              ).astype(f32)
    h = jnp.dot(x_ref[...], w0x_ref[...], preferred_element_type=f32)
    sE = s0_ref[..., :D_EDGE] + s1_ref[..., :D_EDGE]
    h = h + jnp.dot(sE, w0e_ref[...], preferred_element_type=f32)
    h = h + jnp.dot(onehot, uw, preferred_element_type=f32)
    h = jnp.maximum(h + b0_ref[...], 0.0)
    h = jnp.maximum(jnp.dot(h, w1_ref[...], preferred_element_type=f32) + b1_ref[...], 0.0)
    h = jnp.maximum(jnp.dot(h, w2_ref[...], preferred_element_type=f32) + b2_ref[...], 0.0)
    mu = jnp.mean(h, axis=1, keepdims=True)
    var = jnp.mean((h - mu) * (h - mu), axis=1, keepdims=True)
    o_ref[...] = (h - mu) * lax.rsqrt(var + 1e-5) * g_ref[...] + bb_ref[...]


def _mlp_call(x, s0, s1, batch2d, u, W0x, W0e, W0u, b0, W1, b1, W2, b2, g, bb):
    grid = (N_NODES // BN,)
    full = lambda shape: pl.BlockSpec(shape, lambda i: (0, 0))
    return pl.pallas_call(
        _mlp_body,
        grid=grid,
        in_specs=[
            pl.BlockSpec((BN, D_FEAT), lambda i: (i, 0)),
            pl.BlockSpec((BN, 128), lambda i: (i, 0)),
            pl.BlockSpec((BN, 128), lambda i: (i, 0)),
            pl.BlockSpec((BN, 1), lambda i: (i, 0)),
            full((N_GRAPHS, D_GLOBAL)),
            full((D_FEAT, H1)),
            full((D_EDGE, H1)),
            full((D_GLOBAL, H1)),
            full((1, H1)),
            full((H1, H2)),
            full((1, H2)),
            full((H2, N_OUT)),
            full((1, N_OUT)),
            full((1, N_OUT)),
            full((1, N_OUT)),
        ],
        out_specs=pl.BlockSpec((BN, N_OUT), lambda i: (i, 0)),
        out_shape=jax.ShapeDtypeStruct((N_NODES, N_OUT), jnp.float32),
    )(x, s0, s1, batch2d, u, W0x, W0e, W0u, b0, W1, b1, W2, b2, g, bb)


def kernel(x, e, u, edge_index, batch, W0, b0, W1, b1, W2, b2, ln_scale, ln_bias):
    ev = e.T.reshape(2, 8, N_ROWS, EPR).transpose(0, 2, 1, 3)
    ei = edge_index.reshape(2, N_ROWS, EPR).transpose(1, 0, 2)
    s0, s1 = _sc_scatter(ev, ei)
    W0x = W0[:D_FEAT]
    W0e = W0[D_FEAT:D_FEAT + D_EDGE]
    W0u = W0[D_FEAT + D_EDGE:]
    return _mlp_call(
        x, s0, s1, batch[:, None], u, W0x, W0e, W0u, b0[None, :],
        W1, b1[None, :], W2, b2[None, :], ln_scale[None, :], ln_bias[None, :])


# speedup vs baseline: 14.1288x; 1.2743x over previous
import functools

import jax
import jax.numpy as jnp
from jax import lax
from jax.experimental import pallas as pl
from jax.experimental.pallas import tpu as pltpu
from jax.experimental.pallas import tpu_sc as plsc

N_NODES = 10000
N_EDGES = 320000
D_FEAT = 128
D_EDGE = 16
N_GRAPHS = 16
D_GLOBAL = 32
H1 = 256
H2 = 256
N_OUT = 128

BN = 1000

NC = 2
NS = 16
NW = NC * NS
EPR = 128
N_ROWS = N_EDGES // EPR
K_FULL = N_ROWS // NW
N_EXTRA = N_ROWS - K_FULL * NW
NPT = 1000


def _sc_scatter_build():
    mesh = plsc.VectorSubcoreMesh(core_axis_name="c", subcore_axis_name="s")
    NBUF = 6

    @functools.partial(
        pl.kernel, mesh=mesh,
        compiler_params=pltpu.CompilerParams(
            use_tc_tiling_on_sc=False, needs_layout_passes=False),
        out_type=(jax.ShapeDtypeStruct((N_NODES, 128), jnp.float32),
                  jax.ShapeDtypeStruct((N_NODES, 128), jnp.float32)),
        scratch_types=(
            [pltpu.VMEM((EPR,), jnp.int32) for _ in range(NBUF)]
            + [pltpu.VMEM((2, 8, EPR), jnp.float32) for _ in range(NBUF)]
            + [pltpu.VMEM((EPR, D_EDGE), jnp.float32) for _ in range(NBUF)]
            + [pltpu.VMEM((NPT, D_EDGE), jnp.float32),
               pltpu.VMEM_SHARED((N_NODES, D_EDGE), jnp.float32)]
            + [pltpu.SemaphoreType.DMA for _ in range(2 * NBUF)]
        ))
    def sc_scatter(ev_hbm, ei_hbm, out0_hbm, out1_hbm, *bufs):
        idx = bufs[0:NBUF]
        tb = bufs[NBUF:2 * NBUF]
        sb = bufs[2 * NBUF:3 * NBUF]
        stage = bufs[3 * NBUF]
        acc = bufs[3 * NBUF + 1]
        sl = bufs[3 * NBUF + 2:3 * NBUF + 2 + NBUF]
        ss = bufs[3 * NBUF + 2 + NBUF:3 * NBUF + 2 + 2 * NBUF]

        c = lax.axis_index("c")
        s = lax.axis_index("s")
        w = s * NC + c
        iota16 = lax.broadcasted_iota(jnp.int32, (D_EDGE,), 0)

        def zrow(i, carry):
            stage[i, :] = jnp.zeros((D_EDGE,), jnp.float32)
            return carry
        lax.fori_loop(0, NPT, zrow, 0)

        @pl.when(s < N_NODES // NPT)
        def _():
            pltpu.sync_copy(stage, acc.at[pl.ds(s * NPT, NPT)])
        plsc.subcore_barrier()

        def start_load(k, b):
            r = w + k * NW
            pltpu.async_copy(ei_hbm.at[r, 1], idx[b], sl[b])
            pltpu.async_copy(ev_hbm.at[:, r], tb[b], sl[b])

        def wait_load(b):
            pltpu.make_async_copy(ei_hbm.at[0, 1], idx[b], sl[b]).wait()
            pltpu.make_async_copy(ev_hbm.at[:, 0], tb[b], sl[b]).wait()

        rot = [((iota16 + d) & 15) for d in range(D_EDGE)]
        hi3 = iota16 >> 3
        lo3 = iota16 & 7
        zv = jnp.zeros((D_EDGE,), jnp.int32)
        ldb = [hi3 * (8 * EPR) + lo3 * EPR + rot[d] for d in range(D_EDGE)]
        stb = [rot[d] * D_EDGE + iota16 for d in range(D_EDGE)]

        def transpose(b):
            def tr(m, carry):
                vs = [plsc.load_gather(tb[b], [zv, zv, ldb[d] + m * D_EDGE])
                      for d in range(D_EDGE)]
                for d in range(D_EDGE):
                    plsc.store_scatter(
                        sb[b], [zv, stb[d] + m * (D_EDGE * D_EDGE)], vs[d])
                return carry
            lax.fori_loop(0, EPR // D_EDGE, tr, 0)

        def start_scat(b):
            pltpu.async_copy(sb[b], acc.at[idx[b]], ss[b], add=True)

        def wait_scat(b):
            pltpu.make_async_copy(sb[b], acc.at[idx[b]], ss[b]).wait()

        for b in range(3):
            start_load(b, b)

        def body(g, carry):
            for i in range(NBUF):
                b = i
                k = g * NBUF + i

                wait_load(b)
                transpose(b)
                start_scat(b)

                bb = (i + 3) % NBUF

                @pl.when(k + 3 < K_FULL)
                def _():
                    @pl.when(k >= 3)
                    def _():
                        wait_scat(bb)
                    start_load(k + 3, bb)
            return carry

        lax.fori_loop(0, K_FULL // NBUF, body, 0)

        @pl.when(w < N_EXTRA)
        def _():
            wait_scat(0)
            start_load(K_FULL, 0)
            wait_load(0)
            transpose(0)
            start_scat(0)

        for b in range(NBUF):
            wait_scat(b)

        plsc.subcore_barrier()

        @pl.when(s < N_NODES // NPT)
        def _():
            src_slice = acc.at[pl.ds(s * NPT, NPT)]

            @pl.when(c == 0)
            def _():
                pltpu.sync_copy(src_slice,
                                out0_hbm.at[pl.ds(s * NPT, NPT), pl.ds(0, D_EDGE)])

            @pl.when(c == 1)
            def _():
                pltpu.sync_copy(src_slice,
                                out1_hbm.at[pl.ds(s * NPT, NPT), pl.ds(0, D_EDGE)])

    return sc_scatter


_sc_scatter = _sc_scatter_build()


def _mlp_body(x_ref, s0_ref, s1_ref, bat_ref, u_ref, w0x_ref, w0e_ref, w0u_ref,
              b0_ref, w1_ref, b1_ref, w2_ref, b2_ref, g_ref, bb_ref, o_ref):
    f32 = jnp.float32
    uw = jnp.dot(u_ref[...], w0u_ref[...], preferred_element_type=f32)
    onehot = (bat_ref[...] == lax.broadcasted_iota(jnp.int32, (BN, N_GRAPHS), 1)
              ).astype(f32)
    h = jnp.dot(x_ref[...], w0x_ref[...], preferred_element_type=f32)
    sE = s0_ref[..., :D_EDGE] + s1_ref[..., :D_EDGE]
    h = h + jnp.dot(sE, w0e_ref[...], preferred_element_type=f32)
    h = h + jnp.dot(onehot, uw, preferred_element_type=f32)
    h = jnp.maximum(h + b0_ref[...], 0.0)
    h = jnp.maximum(jnp.dot(h, w1_ref[...], preferred_element_type=f32) + b1_ref[...], 0.0)
    h = jnp.maximum(jnp.dot(h, w2_ref[...], preferred_element_type=f32) + b2_ref[...], 0.0)
    mu = jnp.mean(h, axis=1, keepdims=True)
    var = jnp.mean((h - mu) * (h - mu), axis=1, keepdims=True)
    o_ref[...] = (h - mu) * lax.rsqrt(var + 1e-5) * g_ref[...] + bb_ref[...]


def _mlp_call(x, s0, s1, batch2d, u, W0x, W0e, W0u, b0, W1, b1, W2, b2, g, bb):
    grid = (N_NODES // BN,)
    full = lambda shape: pl.BlockSpec(shape, lambda i: (0, 0))
    return pl.pallas_call(
        _mlp_body,
        grid=grid,
        in_specs=[
            pl.BlockSpec((BN, D_FEAT), lambda i: (i, 0)),
            pl.BlockSpec((BN, 128), lambda i: (i, 0)),
            pl.BlockSpec((BN, 128), lambda i: (i, 0)),
            pl.BlockSpec((BN, 1), lambda i: (i, 0)),
            full((N_GRAPHS, D_GLOBAL)),
            full((D_FEAT, H1)),
            full((D_EDGE, H1)),
            full((D_GLOBAL, H1)),
            full((1, H1)),
            full((H1, H2)),
            full((1, H2)),
            full((H2, N_OUT)),
            full((1, N_OUT)),
            full((1, N_OUT)),
            full((1, N_OUT)),
        ],
        out_specs=pl.BlockSpec((BN, N_OUT), lambda i: (i, 0)),
        out_shape=jax.ShapeDtypeStruct((N_NODES, N_OUT), jnp.float32),
    )(x, s0, s1, batch2d, u, W0x, W0e, W0u, b0, W1, b1, W2, b2, g, bb)


def kernel(x, e, u, edge_index, batch, W0, b0, W1, b1, W2, b2, ln_scale, ln_bias):
    ev = e.T.reshape(2, 8, N_ROWS, EPR).transpose(0, 2, 1, 3)
    ei = edge_index.reshape(2, N_ROWS, EPR).transpose(1, 0, 2)
    s0, s1 = _sc_scatter(ev, ei)
    W0x = W0[:D_FEAT]
    W0e = W0[D_FEAT:D_FEAT + D_EDGE]
    W0u = W0[D_FEAT + D_EDGE:]
    return _mlp_call(
        x, s0, s1, batch[:, None], u, W0x, W0e, W0u, b0[None, :],
        W1, b1[None, :], W2, b2[None, :], ln_scale[None, :], ln_bias[None, :])


# speedup vs baseline: 14.6354x; 1.0359x over previous
import functools

import jax
import jax.numpy as jnp
from jax import lax
from jax.experimental import pallas as pl
from jax.experimental.pallas import tpu as pltpu
from jax.experimental.pallas import tpu_sc as plsc

N_NODES = 10000
N_EDGES = 320000
D_FEAT = 128
D_EDGE = 16
N_GRAPHS = 16
D_GLOBAL = 32
H1 = 256
H2 = 256
N_OUT = 128

BN = 2000

NC = 2
NS = 16
NW = NC * NS
EPR = 128
N_ROWS = N_EDGES // EPR
K_FULL = N_ROWS // NW
N_EXTRA = N_ROWS - K_FULL * NW
NPT = 1000


def _sc_scatter_build():
    mesh = plsc.VectorSubcoreMesh(core_axis_name="c", subcore_axis_name="s")
    NBUF = 6

    @functools.partial(
        pl.kernel, mesh=mesh,
        compiler_params=pltpu.CompilerParams(
            use_tc_tiling_on_sc=False, needs_layout_passes=False),
        out_type=(jax.ShapeDtypeStruct((N_NODES, 128), jnp.float32),
                  jax.ShapeDtypeStruct((N_NODES, 128), jnp.float32)),
        scratch_types=(
            [pltpu.VMEM((EPR,), jnp.int32) for _ in range(NBUF)]
            + [pltpu.VMEM((2, 8, EPR), jnp.float32) for _ in range(NBUF)]
            + [pltpu.VMEM((EPR, D_EDGE), jnp.float32) for _ in range(NBUF)]
            + [pltpu.VMEM((NPT, D_EDGE), jnp.float32),
               pltpu.VMEM_SHARED((N_NODES, D_EDGE), jnp.float32)]
            + [pltpu.SemaphoreType.DMA for _ in range(2 * NBUF)]
        ))
    def sc_scatter(ev_hbm, ei_hbm, out0_hbm, out1_hbm, *bufs):
        idx = bufs[0:NBUF]
        tb = bufs[NBUF:2 * NBUF]
        sb = bufs[2 * NBUF:3 * NBUF]
        stage = bufs[3 * NBUF]
        acc = bufs[3 * NBUF + 1]
        sl = bufs[3 * NBUF + 2:3 * NBUF + 2 + NBUF]
        ss = bufs[3 * NBUF + 2 + NBUF:3 * NBUF + 2 + 2 * NBUF]

        c = lax.axis_index("c")
        s = lax.axis_index("s")
        w = s * NC + c
        iota16 = lax.broadcasted_iota(jnp.int32, (D_EDGE,), 0)

        def zrow(i, carry):
            stage[i, :] = jnp.zeros((D_EDGE,), jnp.float32)
            return carry
        lax.fori_loop(0, NPT, zrow, 0)

        @pl.when(s < N_NODES // NPT)
        def _():
            pltpu.sync_copy(stage, acc.at[pl.ds(s * NPT, NPT)])
        plsc.subcore_barrier()

        def start_load(k, b):
            r = w + k * NW
            pltpu.async_copy(ei_hbm.at[r, 1], idx[b], sl[b])
            pltpu.async_copy(ev_hbm.at[:, r], tb[b], sl[b])

        def wait_load(b):
            pltpu.make_async_copy(ei_hbm.at[0, 1], idx[b], sl[b]).wait()
            pltpu.make_async_copy(ev_hbm.at[:, 0], tb[b], sl[b]).wait()

        rot = [((iota16 + d) & 15) for d in range(D_EDGE)]
        hi3 = iota16 >> 3
        lo3 = iota16 & 7
        zv = jnp.zeros((D_EDGE,), jnp.int32)
        ldb = [hi3 * (8 * EPR) + lo3 * EPR + rot[d] for d in range(D_EDGE)]
        stb = [rot[d] * D_EDGE + iota16 for d in range(D_EDGE)]

        def transpose(b):
            def tr(m, carry):
                vs = [plsc.load_gather(tb[b], [zv, zv, ldb[d] + m * D_EDGE])
                      for d in range(D_EDGE)]
                for d in range(D_EDGE):
                    plsc.store_scatter(
                        sb[b], [zv, stb[d] + m * (D_EDGE * D_EDGE)], vs[d])
                return carry
            lax.fori_loop(0, EPR // D_EDGE, tr, 0)

        def start_scat(b):
            pltpu.async_copy(sb[b], acc.at[idx[b]], ss[b], add=True)

        def wait_scat(b):
            pltpu.make_async_copy(sb[b], acc.at[idx[b]], ss[b]).wait()

        for b in range(3):
            start_load(b, b)

        def body(g, carry):
            for i in range(NBUF):
                b = i
                k = g * NBUF + i

                wait_load(b)
                transpose(b)
                start_scat(b)

                bb = (i + 3) % NBUF

                @pl.when(k + 3 < K_FULL)
                def _():
                    @pl.when(k >= 3)
                    def _():
                        wait_scat(bb)
                    start_load(k + 3, bb)
            return carry

        lax.fori_loop(0, K_FULL // NBUF, body, 0)

        @pl.when(w < N_EXTRA)
        def _():
            wait_scat(0)
            start_load(K_FULL, 0)
            wait_load(0)
            transpose(0)
            start_scat(0)

        for b in range(NBUF):
            wait_scat(b)

        plsc.subcore_barrier()

        @pl.when(s < N_NODES // NPT)
        def _():
            src_slice = acc.at[pl.ds(s * NPT, NPT)]

            @pl.when(c == 0)
            def _():
                pltpu.sync_copy(src_slice,
                                out0_hbm.at[pl.ds(s * NPT, NPT), pl.ds(0, D_EDGE)])

            @pl.when(c == 1)
            def _():
                pltpu.sync_copy(src_slice,
                                out1_hbm.at[pl.ds(s * NPT, NPT), pl.ds(0, D_EDGE)])

    return sc_scatter


_sc_scatter = _sc_scatter_build()


def _mlp_body(x_ref, s0_ref, s1_ref, bat_ref, u_ref, w0_ref,
              b0_ref, w1_ref, b1_ref, w2_ref, b2_ref, g_ref, bb_ref, o_ref):
    f32 = jnp.float32
    uw = jnp.dot(u_ref[...], w0_ref[D_FEAT + D_EDGE:, :],
                 preferred_element_type=f32)
    onehot = (bat_ref[...] == lax.broadcasted_iota(jnp.int32, (BN, N_GRAPHS), 1)
              ).astype(f32)
    h = jnp.dot(x_ref[...], w0_ref[:D_FEAT, :], preferred_element_type=f32)
    sE = s0_ref[..., :D_EDGE] + s1_ref[..., :D_EDGE]
    h = h + jnp.dot(sE, w0_ref[D_FEAT:D_FEAT + D_EDGE, :],
                    preferred_element_type=f32)
    h = h + jnp.dot(onehot, uw, preferred_element_type=f32)
    h = jnp.maximum(h + b0_ref[...], 0.0)
    h = jnp.maximum(jnp.dot(h, w1_ref[...], preferred_element_type=f32) + b1_ref[...], 0.0)
    h = jnp.maximum(jnp.dot(h, w2_ref[...], preferred_element_type=f32) + b2_ref[...], 0.0)
    mu = jnp.mean(h, axis=1, keepdims=True)
    var = jnp.mean((h - mu) * (h - mu), axis=1, keepdims=True)
    o_ref[...] = (h - mu) * lax.rsqrt(var + 1e-5) * g_ref[...] + bb_ref[...]


def _mlp_call(x, s0, s1, batch2d, u, W0, b0, W1, b1, W2, b2, g, bb):
    grid = (N_NODES // BN,)
    full = lambda shape: pl.BlockSpec(shape, lambda i: (0, 0))
    return pl.pallas_call(
        _mlp_body,
        grid=grid,
        in_specs=[
            pl.BlockSpec((BN, D_FEAT), lambda i: (i, 0)),
            pl.BlockSpec((BN, 128), lambda i: (i, 0)),
            pl.BlockSpec((BN, 128), lambda i: (i, 0)),
            pl.BlockSpec((BN, 1), lambda i: (i, 0)),
            full((N_GRAPHS, D_GLOBAL)),
            full((D_FEAT + D_EDGE + D_GLOBAL, H1)),
            full((1, H1)),
            full((H1, H2)),
            full((1, H2)),
            full((H2, N_OUT)),
            full((1, N_OUT)),
            full((1, N_OUT)),
            full((1, N_OUT)),
        ],
        out_specs=pl.BlockSpec((BN, N_OUT), lambda i: (i, 0)),
        out_shape=jax.ShapeDtypeStruct((N_NODES, N_OUT), jnp.float32),
    )(x, s0, s1, batch2d, u, W0, b0, W1, b1, W2, b2, g, bb)


def kernel(x, e, u, edge_index, batch, W0, b0, W1, b1, W2, b2, ln_scale, ln_bias):
    ev = e.T.reshape(2, 8, N_ROWS, EPR).transpose(0, 2, 1, 3)
    ei = edge_index.reshape(2, N_ROWS, EPR).transpose(1, 0, 2)
    s0, s1 = _sc_scatter(ev, ei)
    return _mlp_call(
        x, s0, s1, batch[:, None], u, W0, b0[None, :],
        W1, b1[None, :], W2, b2[None, :], ln_scale[None, :], ln_bias[None, :])
